# baseline (speedup 1.0000x reference)
import functools

import jax
import jax.numpy as jnp
from jax import lax
from jax.experimental import pallas as pl
from jax.experimental.pallas import tpu as pltpu
from jax.experimental.pallas import tpu_sc as plsc

DIM = 64
NW = 32
SUB = 128
XCOL = 20


def _dequant_gather(q8, aux4, idx2, idx4, n1, n_flat):
    rows_per_w = n_flat // NW
    chunk = 640
    x_per_chunk = chunk // XCOL
    nchunks = rows_per_w // chunk
    nsub = chunk // SUB
    mesh = plsc.VectorSubcoreMesh(core_axis_name="c", subcore_axis_name="s")

    @functools.partial(
        pl.kernel,
        mesh=mesh,
        out_type=jax.ShapeDtypeStruct((n1, XCOL, DIM), jnp.float32),
        compiler_params=pltpu.CompilerParams(
            needs_layout_passes=False, use_tc_tiling_on_sc=False),
        scratch_types=[
            pltpu.VMEM((nsub, SUB), jnp.int32),
            pltpu.VMEM((nsub, SUB), jnp.int32),
            pltpu.VMEM((chunk, DIM), jnp.int8),
            pltpu.VMEM((chunk, 16), jnp.float32),
            pltpu.VMEM((x_per_chunk, XCOL, DIM), jnp.float32),
            pltpu.SemaphoreType.DMA,
        ],
    )
    def body(q8_ref, aux_ref, idx_ref, idx4_ref, out_ref,
             idx_v, idx4_v, q_v, a_v, out_v, sem):
        wid = lax.axis_index("s") * 2 + lax.axis_index("c")
        lanes = lax.iota(jnp.int32, 16)
        gsel = lanes >> 3
        ccols = [lanes * 4 + k for k in range(4)]

        for c in range(nchunks):
            base = wid * rows_per_w + c * chunk
            ib = wid * (rows_per_w // SUB) + c * nsub
            pltpu.sync_copy(idx_ref.at[pl.ds(ib, nsub)], idx_v)
            pltpu.sync_copy(idx4_ref.at[pl.ds(ib, nsub)], idx4_v)
            copies = []
            for j in range(nsub):
                copies.append(pltpu.async_copy(
                    q8_ref.at[idx_v.at[j]],
                    q_v.at[pl.ds(j * SUB, SUB)], sem))
                copies.append(pltpu.async_copy(
                    aux_ref.at[idx4_v.at[j]],
                    a_v.at[pl.ds(j * SUB, SUB)], sem))
            for cp in copies:
                cp.wait()

            def n_body(nl, carry):

                def b_body(bb, carry2):
                    r = nl * XCOL + bb
                    rsp = jnp.full((16,), r, jnp.int32)
                    vi = plsc.load_gather(
                        idx_v, [jnp.full((16,), r >> 7, jnp.int32),
                                jnp.full((16,), r & 127, jnp.int32)])
                    acol = ((vi & 3) << 2) + gsel
                    sv = plsc.load_gather(a_v, [rsp, acol])
                    zv = plsc.load_gather(a_v, [rsp, acol + 2])
                    qw = plsc.bitcast(q_v[r, :], jnp.int32)
                    nsp = jnp.full((16,), nl, jnp.int32)
                    bsp = jnp.full((16,), bb, jnp.int32)
                    for k in range(4):
                        pk = (qw << (24 - 8 * k)) >> 24 if k < 3 else qw >> 24
                        res = (pk.astype(jnp.float32) - zv) * sv
                        plsc.store_scatter(out_v, [nsp, bsp, ccols[k]], res)
                    return carry2

                return lax.fori_loop(0, XCOL, b_body, carry)

            lax.fori_loop(0, x_per_chunk, n_body, 0)
            pltpu.sync_copy(
                out_v, out_ref.at[pl.ds(base // XCOL, x_per_chunk)])

    return body(q8, aux4, idx2, idx4)


def kernel(packed_weight_qvals, weight_scales, weight_zeros, x):
    V, D = packed_weight_qvals.shape
    aux4 = jnp.concatenate(
        [weight_scales, weight_zeros.astype(jnp.float32)],
        axis=1).reshape(V // 4, 16)
    flat = x.reshape(-1).astype(jnp.int32)
    n_flat = flat.shape[0]
    idx2 = flat.reshape(n_flat // SUB, SUB)
    idx4 = (flat >> 2).reshape(n_flat // SUB, SUB)
    out = _dequant_gather(packed_weight_qvals, aux4, idx2, idx4,
                          x.shape[0], n_flat)
    return out.reshape(*x.shape, D)
